# baseline (speedup 1.0000x reference)
import functools

import jax
import jax.numpy as jnp
from jax import lax
from jax.experimental import pallas as pl
from jax.experimental.pallas import tpu as pltpu
from jax.experimental.pallas import tpu_sc as plsc

N = 100000
E = 6400000
DIM = 128
G = 256

_NC, _NS = 2, 16
_TILES = _NC * _NS
_C = 128
_K = 16
_CPT = 1568
_NOUT = _CPT // _K
_ROWS = _TILES * _CPT
_EP = _ROWS * _C
_D = 16
_ZR = 3136
_AGGR = _ZR * _NS
_XROWS = 200016

_TW = 8
_BN = 2000
_NB = N // _BN


def _sc_agg_build():
    mesh = plsc.VectorSubcoreMesh(core_axis_name="c", subcore_axis_name="s",
                                  num_cores=_NC, num_subcores=_NS)

    @functools.partial(
        pl.kernel,
        out_type=jax.ShapeDtypeStruct((_NC, _AGGR, _D), jnp.float32),
        mesh=mesh,
        scratch_types=[
            pltpu.VMEM((_K, _C), jnp.int32),
            pltpu.VMEM((_K, _C), jnp.int32),
            pltpu.VMEM((_K, _C, _D), jnp.float32),
            pltpu.VMEM_SHARED((_AGGR, _D), jnp.float32),
            pltpu.SemaphoreType.DMA,
            pltpu.SemaphoreType.DMA,
        ],
        compiler_params=pltpu.CompilerParams(use_tc_tiling_on_sc=False),
    )
    def sc_agg(gidx_hbm, sidx_hbm, xtab_hbm, z_hbm, out_hbm,
               sbuf, dbuf, rbuf, aggsh, gsem, ssem):
        c = lax.axis_index("c")
        s = lax.axis_index("s")
        tid = s * _NC + c
        pltpu.sync_copy(z_hbm.at[pl.ds(s * _ZR, _ZR)],
                        aggsh.at[pl.ds(s * _ZR, _ZR)])
        plsc.subcore_barrier()

        def body(g, carry):
            rowbase = tid * _CPT + g * _K
            pltpu.sync_copy(gidx_hbm.at[pl.ds(rowbase, _K)], sbuf)
            pltpu.sync_copy(sidx_hbm.at[pl.ds(rowbase, _K)], dbuf)
            gs = [pltpu.async_copy(xtab_hbm.at[sbuf.at[j]], rbuf.at[j], gsem)
                  for j in range(_K)]
            for cp in gs:
                cp.wait()
            ss = [pltpu.async_copy(rbuf.at[j], aggsh.at[dbuf.at[j]], ssem,
                                   add=True)
                  for j in range(_K)]
            for cp in ss:
                cp.wait()
            return carry

        lax.fori_loop(0, _NOUT, body, 0)
        plsc.subcore_barrier()
        pltpu.sync_copy(aggsh.at[pl.ds(s * _ZR, _ZR)],
                        out_hbm.at[c, pl.ds(s * _ZR, _ZR)])

    return sc_agg


def _sc_agg_call(gidx, sidx, xtab, zeros16):
    return _sc_agg_build()(gidx, sidx, xtab, zeros16)


def _tc_body(x8_ref, agg_ref, bat_ref,
             W1_ref, b1_ref, W2_ref, b2_ref, W3_ref, b3_ref,
             Wg_ref, bg_ref, Wf_ref, bf_ref, gb_ref, bb_ref,
             out_ref, sums, sq, M, P, D):
    p = pl.program_id(0)
    i = pl.program_id(1)
    h0 = x8_ref[...] + agg_ref[0] + agg_ref[1]

    @pl.when((p == 0) & (i == 0))
    def _init():
        sums[...] = jnp.zeros_like(sums)
        sq[...] = jnp.zeros_like(sq)
        M[...] = jnp.full_like(M, -jnp.inf)
        P[...] = jnp.zeros_like(P)
        D[...] = jnp.zeros_like(D)

    @pl.when(p == 0)
    def _stats():
        sums[...] += jnp.sum(h0, axis=0, keepdims=True)
        sq[...] += jnp.sum(h0 * h0, axis=0, keepdims=True)

    @pl.when(p == 1)
    def _mlp():
        mean = sums[...] * (1.0 / N)
        var = sq[...] * (1.0 / N) - mean * mean
        hbn = (h0 - mean) * (gb_ref[...] * lax.rsqrt(var + 1e-5)) + bb_ref[...]
        h1 = jnp.maximum(jnp.dot(hbn, W1_ref[...],
                                 preferred_element_type=jnp.float32)
                         + b1_ref[...], 0.0)
        h2 = jnp.maximum(jnp.dot(h1, W2_ref[...],
                                 preferred_element_type=jnp.float32)
                         + b2_ref[...], 0.0)
        h3 = jnp.maximum(jnp.dot(h2, W3_ref[...],
                                 preferred_element_type=jnp.float32)
                         + b3_ref[...], 0.0)
        gate = jnp.dot(h3, Wg_ref[...],
                       preferred_element_type=jnp.float32) + bg_ref[...]
        oh = bat_ref[...] == lax.broadcasted_iota(jnp.int32, (_BN, G), 1)
        ohf = oh.astype(jnp.float32)
        m_b = jnp.max(jnp.where(oh, gate, -jnp.inf), axis=0, keepdims=True)
        m_new = jnp.maximum(M[...], m_b)
        scl = jnp.where(m_new == -jnp.inf, 1.0, jnp.exp(M[...] - m_new))
        m_node = jnp.max(jnp.where(oh, m_new, -jnp.inf), axis=1, keepdims=True)
        e = jnp.exp(gate - m_node)
        he = h3 * e
        dn = (((0,), (0,)), ((), ()))
        P[...] = P[...] * scl + lax.dot_general(
            he, ohf, dn, preferred_element_type=jnp.float32)
        D[...] = D[...] * scl + lax.dot_general(
            e, ohf, dn, preferred_element_type=jnp.float32)
        M[...] = m_new

    @pl.when((p == 1) & (i == _NB - 1))
    def _final():
        pooled = P[...] / jnp.maximum(D[...], 1e-16)
        out_ref[...] = lax.dot_general(
            pooled, Wf_ref[...], (((0,), (0,)), ((), ())),
            preferred_element_type=jnp.float32) + bf_ref[...]


def _tc_call(x8p, aggs8, batch_s, W1p, b1, W2, b2, W3, b3, Wg, bg, Wf, bf,
             gb8, bb8):
    f32 = jnp.float32
    full = lambda shape: pl.BlockSpec(shape, lambda p, i: tuple(0 for _ in shape))
    return pl.pallas_call(
        _tc_body,
        grid=(2, _NB),
        in_specs=[
            pl.BlockSpec((_BN, _TW), lambda p, i: (i, 0)),
            pl.BlockSpec((_NC, _BN, _TW), lambda p, i: (0, i, 0)),
            pl.BlockSpec((_BN, 1), lambda p, i: (i, 0)),
            full((_TW, DIM)), full((1, DIM)),
            full((DIM, DIM)), full((1, DIM)),
            full((DIM, DIM)), full((1, DIM)),
            full((DIM, 1)), full((1, 1)),
            full((DIM, 1)), full((1, 1)),
            full((1, _TW)), full((1, _TW)),
        ],
        out_specs=pl.BlockSpec((G, 1), lambda p, i: (0, 0)),
        out_shape=jax.ShapeDtypeStruct((G, 1), f32),
        scratch_shapes=[
            pltpu.VMEM((1, _TW), f32),
            pltpu.VMEM((1, _TW), f32),
            pltpu.VMEM((1, G), f32),
            pltpu.VMEM((DIM, G), f32),
            pltpu.VMEM((1, G), f32),
        ],
    )(x8p, aggs8, batch_s, W1p, b1, W2, b2, W3, b3, Wg, bg, Wf, bf, gb8, bb8)


def kernel(x, edge_index, batch, base, bn_gamma, bn_beta,
           W1, b1, W2, b2, W3, b3, Wg, bg, Wf, bf):
    f32 = jnp.float32
    pad_e = _EP - E
    src = edge_index[0]
    dst = edge_index[1]
    gidx = jnp.concatenate(
        [2 * src + (dst & 1),
         jnp.full((pad_e,), 2 * N, jnp.int32)]).reshape(_ROWS, _C)
    sidx = jnp.concatenate(
        [dst >> 1, jnp.zeros((pad_e,), jnp.int32)]).reshape(_ROWS, _C)
    xtab = (jnp.zeros((_XROWS, _D), f32)
            .at[0:2 * N:2, 0:3].set(x)
            .at[1:2 * N + 1:2, 8:11].set(x))
    zeros16 = jnp.zeros((_AGGR, _D), f32)

    aggs = _sc_agg_call(gidx, sidx, xtab, zeros16)
    aggs8 = aggs.reshape(_NC, 2 * _AGGR, _TW)

    batch_s = batch.reshape(N, 1)
    x8p = jnp.zeros((N, _TW), f32).at[:, :3].set(x)
    W1p = jnp.zeros((_TW, DIM), f32).at[:3].set(W1)
    gb8 = jnp.ones((1, _TW), f32).at[0, :3].set(bn_gamma)
    bb8 = jnp.zeros((1, _TW), f32).at[0, :3].set(bn_beta)
    out = _tc_call(x8p, aggs8, batch_s,
                   W1p, b1.reshape(1, DIM), W2, b2.reshape(1, DIM),
                   W3, b3.reshape(1, DIM), Wg, bg.reshape(1, 1),
                   Wf, bf.reshape(1, 1), gb8, bb8)
    return out


# speedup vs baseline: 1.0018x; 1.0018x over previous
import functools

import jax
import jax.numpy as jnp
from jax import lax
from jax.experimental import pallas as pl
from jax.experimental.pallas import tpu as pltpu
from jax.experimental.pallas import tpu_sc as plsc

N = 100000
E = 6400000
DIM = 128
G = 256

_NC, _NS = 2, 16
_TILES = _NC * _NS
_B = 4096
_NOUT = 49
_ROWS = _TILES * _NOUT
_EP = _ROWS * _B
_D = 16
_ZR = 3136
_AGGR = _ZR * _NS
_XROWS = 200016

_TW = 8
_BN = 2000
_NB = N // _BN


def _sc_agg_build():
    mesh = plsc.VectorSubcoreMesh(core_axis_name="c", subcore_axis_name="s",
                                  num_cores=_NC, num_subcores=_NS)

    @functools.partial(
        pl.kernel,
        out_type=jax.ShapeDtypeStruct((_NC, _AGGR, _D), jnp.float32),
        mesh=mesh,
        scratch_types=[
            pltpu.VMEM((_B,), jnp.int32),
            pltpu.VMEM((_B,), jnp.int32),
            pltpu.VMEM((_B, _D), jnp.float32),
            pltpu.VMEM_SHARED((_AGGR, _D), jnp.float32),
            pltpu.SemaphoreType.DMA,
            pltpu.SemaphoreType.DMA,
        ],
        compiler_params=pltpu.CompilerParams(use_tc_tiling_on_sc=False),
    )
    def sc_agg(gidx_hbm, sidx_hbm, xtab_hbm, z_hbm, out_hbm,
               sbuf, dbuf, rbuf, aggsh, gsem, ssem):
        c = lax.axis_index("c")
        s = lax.axis_index("s")
        tid = s * _NC + c
        pltpu.sync_copy(z_hbm.at[pl.ds(s * _ZR, _ZR)],
                        aggsh.at[pl.ds(s * _ZR, _ZR)])
        plsc.subcore_barrier()

        def body(g, carry):
            row = tid * _NOUT + g
            pltpu.sync_copy(gidx_hbm.at[row], sbuf)
            pltpu.sync_copy(sidx_hbm.at[row], dbuf)
            pltpu.async_copy(xtab_hbm.at[sbuf], rbuf, gsem).wait()
            pltpu.async_copy(rbuf, aggsh.at[dbuf], ssem, add=True).wait()
            return carry

        lax.fori_loop(0, _NOUT, body, 0)
        plsc.subcore_barrier()
        pltpu.sync_copy(aggsh.at[pl.ds(s * _ZR, _ZR)],
                        out_hbm.at[c, pl.ds(s * _ZR, _ZR)])

    return sc_agg


def _sc_agg_call(gidx, sidx, xtab, zeros16):
    return _sc_agg_build()(gidx, sidx, xtab, zeros16)


def _tc_body(x8_ref, agg_ref, bat_ref,
             W1_ref, b1_ref, W2_ref, b2_ref, W3_ref, b3_ref,
             Wg_ref, bg_ref, Wf_ref, bf_ref, gb_ref, bb_ref,
             out_ref, sums, sq, M, P, D):
    p = pl.program_id(0)
    i = pl.program_id(1)
    h0 = x8_ref[...] + agg_ref[0] + agg_ref[1]

    @pl.when((p == 0) & (i == 0))
    def _init():
        sums[...] = jnp.zeros_like(sums)
        sq[...] = jnp.zeros_like(sq)
        M[...] = jnp.full_like(M, -jnp.inf)
        P[...] = jnp.zeros_like(P)
        D[...] = jnp.zeros_like(D)

    @pl.when(p == 0)
    def _stats():
        sums[...] += jnp.sum(h0, axis=0, keepdims=True)
        sq[...] += jnp.sum(h0 * h0, axis=0, keepdims=True)

    @pl.when(p == 1)
    def _mlp():
        mean = sums[...] * (1.0 / N)
        var = sq[...] * (1.0 / N) - mean * mean
        hbn = (h0 - mean) * (gb_ref[...] * lax.rsqrt(var + 1e-5)) + bb_ref[...]
        h1 = jnp.maximum(jnp.dot(hbn, W1_ref[...],
                                 preferred_element_type=jnp.float32)
                         + b1_ref[...], 0.0)
        h2 = jnp.maximum(jnp.dot(h1, W2_ref[...],
                                 preferred_element_type=jnp.float32)
                         + b2_ref[...], 0.0)
        h3 = jnp.maximum(jnp.dot(h2, W3_ref[...],
                                 preferred_element_type=jnp.float32)
                         + b3_ref[...], 0.0)
        gate = jnp.dot(h3, Wg_ref[...],
                       preferred_element_type=jnp.float32) + bg_ref[...]
        oh = bat_ref[...] == lax.broadcasted_iota(jnp.int32, (_BN, G), 1)
        ohf = oh.astype(jnp.float32)
        m_b = jnp.max(jnp.where(oh, gate, -jnp.inf), axis=0, keepdims=True)
        m_new = jnp.maximum(M[...], m_b)
        scl = jnp.where(m_new == -jnp.inf, 1.0, jnp.exp(M[...] - m_new))
        m_node = jnp.max(jnp.where(oh, m_new, -jnp.inf), axis=1, keepdims=True)
        e = jnp.exp(gate - m_node)
        he = h3 * e
        dn = (((0,), (0,)), ((), ()))
        P[...] = P[...] * scl + lax.dot_general(
            he, ohf, dn, preferred_element_type=jnp.float32)
        D[...] = D[...] * scl + lax.dot_general(
            e, ohf, dn, preferred_element_type=jnp.float32)
        M[...] = m_new

    @pl.when((p == 1) & (i == _NB - 1))
    def _final():
        pooled = P[...] / jnp.maximum(D[...], 1e-16)
        out_ref[...] = lax.dot_general(
            pooled, Wf_ref[...], (((0,), (0,)), ((), ())),
            preferred_element_type=jnp.float32) + bf_ref[...]


def _tc_call(x8p, aggs8, batch_s, W1p, b1, W2, b2, W3, b3, Wg, bg, Wf, bf,
             gb8, bb8):
    f32 = jnp.float32
    full = lambda shape: pl.BlockSpec(shape, lambda p, i: tuple(0 for _ in shape))
    return pl.pallas_call(
        _tc_body,
        grid=(2, _NB),
        in_specs=[
            pl.BlockSpec((_BN, _TW), lambda p, i: (i, 0)),
            pl.BlockSpec((_NC, _BN, _TW), lambda p, i: (0, i, 0)),
            pl.BlockSpec((_BN, 1), lambda p, i: (i, 0)),
            full((_TW, DIM)), full((1, DIM)),
            full((DIM, DIM)), full((1, DIM)),
            full((DIM, DIM)), full((1, DIM)),
            full((DIM, 1)), full((1, 1)),
            full((DIM, 1)), full((1, 1)),
            full((1, _TW)), full((1, _TW)),
        ],
        out_specs=pl.BlockSpec((G, 1), lambda p, i: (0, 0)),
        out_shape=jax.ShapeDtypeStruct((G, 1), f32),
        scratch_shapes=[
            pltpu.VMEM((1, _TW), f32),
            pltpu.VMEM((1, _TW), f32),
            pltpu.VMEM((1, G), f32),
            pltpu.VMEM((DIM, G), f32),
            pltpu.VMEM((1, G), f32),
        ],
    )(x8p, aggs8, batch_s, W1p, b1, W2, b2, W3, b3, Wg, bg, Wf, bf, gb8, bb8)


def kernel(x, edge_index, batch, base, bn_gamma, bn_beta,
           W1, b1, W2, b2, W3, b3, Wg, bg, Wf, bf):
    f32 = jnp.float32
    pad_e = _EP - E
    src = edge_index[0]
    dst = edge_index[1]
    gidx = jnp.concatenate(
        [2 * src + (dst & 1),
         jnp.full((pad_e,), 2 * N, jnp.int32)]).reshape(_ROWS, _B)
    sidx = jnp.concatenate(
        [dst >> 1, jnp.zeros((pad_e,), jnp.int32)]).reshape(_ROWS, _B)
    xtab = (jnp.zeros((_XROWS, _D), f32)
            .at[0:2 * N:2, 0:3].set(x)
            .at[1:2 * N + 1:2, 8:11].set(x))
    zeros16 = jnp.zeros((_AGGR, _D), f32)

    aggs = _sc_agg_call(gidx, sidx, xtab, zeros16)
    aggs8 = aggs.reshape(_NC, 2 * _AGGR, _TW)

    batch_s = batch.reshape(N, 1)
    x8p = jnp.zeros((N, _TW), f32).at[:, :3].set(x)
    W1p = jnp.zeros((_TW, DIM), f32).at[:3].set(W1)
    gb8 = jnp.ones((1, _TW), f32).at[0, :3].set(bn_gamma)
    bb8 = jnp.zeros((1, _TW), f32).at[0, :3].set(bn_beta)
    out = _tc_call(x8p, aggs8, batch_s,
                   W1p, b1.reshape(1, DIM), W2, b2.reshape(1, DIM),
                   W3, b3.reshape(1, DIM), Wg, bg.reshape(1, 1),
                   Wf, bf.reshape(1, 1), gb8, bb8)
    return out
